# baseline (speedup 1.0000x reference)
import functools

import jax
import jax.numpy as jnp
from jax import lax
from jax.experimental import pallas as pl
from jax.experimental.pallas import tpu as pltpu
from jax.experimental.pallas import tpu_sc as plsc

N_USER, N_ITEM, N_ATT, N_INTER = 3000, 5000, 1000, 1000
N_NODES = N_USER + N_ITEM + N_ATT + N_INTER
E = 320000
D = 128
NPAD = 10112
NC, NS = 2, 16
NW = NC * NS
CH = 120
CW = 84
EPW = E // NW
ROWS_PT = NPAD // NS
TP = 96

_mesh = plsc.VectorSubcoreMesh(core_axis_name="c", subcore_axis_name="s")


def _zero_rows(zbuf, zrows, acc_sh, s):
    zero16 = jnp.zeros((16,), jnp.float32)

    def zbody(i, _):
        for g in range(D // 16):
            zbuf[i, pl.ds(g * 16, 16)] = zero16
        return 0
    lax.fori_loop(0, zrows, zbody, 0, unroll=False)
    full, rem = ROWS_PT // zrows, ROWS_PT % zrows
    for j in range(full):
        pltpu.sync_copy(zbuf.at[pl.ds(0, zrows)],
                        acc_sh.at[pl.ds(s * ROWS_PT + j * zrows, zrows)])
    if rem:
        pltpu.sync_copy(zbuf.at[pl.ds(0, rem)],
                        acc_sh.at[pl.ds(s * ROWS_PT + full * zrows, rem)])


@functools.partial(
    pl.kernel,
    out_type=(
        jax.ShapeDtypeStruct((NC * NPAD, D), jnp.float32),
        jax.ShapeDtypeStruct((NW * 160, D), jnp.float32),
    ),
    mesh=_mesh,
    scratch_types=(
        pltpu.VMEM((CW, 2, CH), jnp.int32),
        pltpu.VMEM((2, 80), jnp.int32),
        pltpu.VMEM((CH, D), jnp.float32),
        pltpu.VMEM((80, D), jnp.float32),
        pltpu.VMEM_SHARED((NPAD, D), jnp.float32),
        pltpu.SemaphoreType.DMA,
        pltpu.SemaphoreType.DMA,
    ),
)
def _sc_deg_time(sd_hbm, tidx_hbm, p_hbm, ones_hbm,
                 deg_out, g_out, idxv, tiv, onesv, gbuf, deg_sh,
                 scsem, sem):
    c = lax.axis_index("c")
    s = lax.axis_index("s")
    wid = s * NC + c

    pltpu.sync_copy(tidx_hbm.at[wid], tiv)
    pltpu.sync_copy(ones_hbm, onesv)
    _zero_rows(gbuf, 80, deg_sh, s)
    pltpu.sync_copy(sd_hbm.at[wid], idxv)
    plsc.subcore_barrier()

    def issue(k, _):
        pltpu.async_copy(onesv, deg_sh.at[idxv.at[k, 1]], scsem, add=True)
        return 0
    lax.fori_loop(0, CW, issue, 0, unroll=False)

    for j in range(2):
        pltpu.async_copy(p_hbm.at[tiv.at[j]], gbuf, sem).wait()
        pltpu.sync_copy(gbuf, g_out.at[pl.ds(wid * 160 + j * 80, 80)])

    def drain(k, _):
        pltpu.make_async_copy(onesv, deg_sh.at[idxv.at[k, 1]], scsem).wait()
        return 0
    lax.fori_loop(0, CW, drain, 0, unroll=False)

    plsc.subcore_barrier()
    pltpu.sync_copy(deg_sh.at[pl.ds(s * ROWS_PT, ROWS_PT)],
                    deg_out.at[pl.ds(c * NPAD + s * ROWS_PT, ROWS_PT)])


@functools.partial(
    pl.kernel,
    out_type=jax.ShapeDtypeStruct((NC * NPAD, D), jnp.float32),
    mesh=_mesh,
    scratch_types=(
        pltpu.VMEM((6, 2, CH), jnp.int32),
        pltpu.VMEM((3, CH, D), jnp.float32),
        pltpu.VMEM_SHARED((NPAD, D), jnp.float32),
        [pltpu.SemaphoreType.DMA] * 6,
        [pltpu.SemaphoreType.DMA] * 3,
        [pltpu.SemaphoreType.DMA] * 3,
    ),
)
def _sc_propagate(h_hbm, sd_hbm,
                  parts_out, idxv, rbuf, acc_sh, isems, gsems, scsems):
    c = lax.axis_index("c")
    s = lax.axis_index("s")
    wid = s * NC + c

    _zero_rows(rbuf.at[0], CH, acc_sh, s)
    for t in range(3):
        pltpu.async_copy(sd_hbm.at[wid, t], idxv.at[t], isems[t])
    plsc.subcore_barrier()

    def sub_step(k, t, full):
        s6, b = t % 6, t % 3
        pltpu.make_async_copy(sd_hbm.at[wid, 0], idxv.at[s6],
                              isems[s6]).wait()
        if full or t >= 3:
            pltpu.make_async_copy(rbuf.at[b], acc_sh.at[idxv.at[s6, 1]],
                                  scsems[b]).wait()
        pltpu.async_copy(h_hbm.at[idxv.at[s6, 0]], rbuf.at[b], gsems[b])
        pltpu.async_copy(sd_hbm.at[wid, lax.rem(k + 3, CW)],
                         idxv.at[(s6 + 3) % 6], isems[(s6 + 3) % 6])
        if full or t >= 2:
            b2, s62 = (t + 1) % 3, (t + 4) % 6
            pltpu.make_async_copy(h_hbm.at[idxv.at[s62, 0]], rbuf.at[b2],
                                  gsems[b2]).wait()
            pltpu.async_copy(rbuf.at[b2], acc_sh.at[idxv.at[s62, 1]],
                             scsems[b2], add=True)

    for t in range(6):
        sub_step(jnp.int32(t), t, full=False)

    def sextet(j, _):
        kb = 6 + 6 * j
        for t in range(6):
            sub_step(kb + t, t, full=True)
        return 0
    lax.fori_loop(0, (CW - 6) // 6, sextet, 0, unroll=False)

    for (ksc, b2, s62) in ((CW - 2, (CW - 2) % 3, (CW - 2) % 6),
                           (CW - 1, (CW - 1) % 3, (CW - 1) % 6)):
        pltpu.make_async_copy(h_hbm.at[idxv.at[s62, 0]], rbuf.at[b2],
                              gsems[b2]).wait()
        pltpu.async_copy(rbuf.at[b2], acc_sh.at[idxv.at[s62, 1]],
                         scsems[b2], add=True)
    for b in range(3):
        pltpu.make_async_copy(rbuf.at[b], acc_sh.at[idxv.at[b, 1]],
                              scsems[b]).wait()
    for t in range(3):
        pltpu.make_async_copy(sd_hbm.at[wid, 0], idxv.at[t],
                              isems[t]).wait()

    plsc.subcore_barrier()
    pltpu.sync_copy(acc_sh.at[pl.ds(s * ROWS_PT, ROWS_PT)],
                    parts_out.at[pl.ds(c * NPAD + s * ROWS_PT, ROWS_PT)])


def _tc_ptables(te_ref, tth_ref, out_ref):
    for t in range(5):
        out_ref[t] = jnp.dot(te_ref[...], tth_ref[t],
                             preferred_element_type=jnp.float32,
                             precision=lax.Precision.HIGHEST)


def _rdeg(deg_ref):
    deg = deg_ref[0, :, 0:1] + deg_ref[1, :, 0:1]
    return lax.rsqrt(jnp.maximum(deg, 1.0))


def _assemble_x0(x0b_ref, g_ref):
    g = g_ref[...]
    x0t = (g[0:1000] + g[1000:2000] + g[2000:3000]
           + g[3000:4000] + g[4000:5000])
    x0 = x0b_ref[...]
    return jnp.concatenate(
        [x0[:N_USER + N_ITEM + N_ATT],
         x0[N_USER + N_ITEM + N_ATT:N_NODES] + x0t,
         x0[N_NODES:]], axis=0)


def _tc_h0(x0b_ref, g_ref, deg_ref, w_ref, h0_ref):
    h0_ref[...] = jnp.dot(_assemble_x0(x0b_ref, g_ref), w_ref[...],
                          preferred_element_type=jnp.float32,
                          precision=lax.Precision.HIGHEST) * _rdeg(deg_ref)


def _tc_x0_rdegb(x0b_ref, g_ref, deg_ref, x0_ref, rdegb_ref):
    x0_ref[...] = _assemble_x0(x0b_ref, g_ref)
    rdegb_ref[...] = jnp.broadcast_to(_rdeg(deg_ref), (NPAD, D))


def _leaky_next(parts, rdeg, b, row0, nrows):
    x = rdeg * (parts[0] + parts[1]) + b
    x = jnp.where(x >= 0, x, 0.01 * x)
    mask = (row0 + lax.broadcasted_iota(jnp.int32, (nrows, 1), 0)) < N_NODES
    return jnp.where(mask, x, 0.0)


_BLK = NPAD // 8


def _tc_x1_h1(parts_ref, rdegb_ref, b_ref, w_ref, x1_ref, h1_ref):
    i = pl.program_id(0)
    rdeg = rdegb_ref[...]
    x1 = _leaky_next(parts_ref[...], rdeg, b_ref[...], i * _BLK, _BLK)
    x1_ref[...] = x1
    h1_ref[...] = jnp.dot(x1, w_ref[...],
                          preferred_element_type=jnp.float32,
                          precision=lax.Precision.HIGHEST) * rdeg


def _tc_final(parts_ref, rdegb_ref, b_ref, x0_ref, x1_ref, out_ref):
    i = pl.program_id(0)
    x2 = _leaky_next(parts_ref[...], rdegb_ref[...], b_ref[...],
                     i * 1000, 1000)
    out_ref[...] = (x0_ref[...] + x1_ref[...] + x2) * (1.0 / 3.0)


def kernel(edge_index, user_embed, item_embed, att_embed, time_embed,
           time_node, time_to_hidden, W0, b0, W1, b1):
    f32, i32 = jnp.float32, jnp.int32

    src = edge_index[0].astype(i32).reshape(NW, EPW)
    dst = edge_index[1].astype(i32).reshape(NW, EPW)
    npad_e = CW * CH - EPW
    pad_idx = (N_NODES + (jnp.arange(npad_e, dtype=i32) % (NPAD - N_NODES)))
    pad_tile = jnp.tile(pad_idx[None], (NW, 1))
    src_idx = jnp.concatenate([src, pad_tile], axis=1).reshape(NW, CW, CH)
    dst_idx = jnp.concatenate([dst, pad_tile], axis=1).reshape(NW, CW, CH)
    sd_idx = jnp.stack([src_idx, dst_idx], axis=2)

    tn = time_node.astype(i32)
    tidx = (tn.T + TP * jnp.arange(5, dtype=i32)[:, None]).reshape(-1)
    pad_t = 94 + TP * (jnp.arange(NW * 160 - 5 * N_INTER, dtype=i32) % 5)
    tidx = jnp.concatenate([tidx, pad_t]).reshape(NW, 2, 80)

    te_pad = jnp.zeros((TP, D), f32).at[:94, :16].set(time_embed)
    tth_pad = jnp.zeros((5, D, D), f32).at[:, :16, :].set(
        time_to_hidden.reshape(5, 16, D))

    ones128 = jnp.ones((CH, D), f32)

    x0_base = jnp.concatenate(
        [user_embed, item_embed, att_embed,
         jnp.zeros((NPAD - N_USER - N_ITEM - N_ATT, D), f32)], axis=0)

    p_tables = pl.pallas_call(
        _tc_ptables,
        out_shape=jax.ShapeDtypeStruct((5, TP, D), f32),
    )(te_pad, tth_pad)
    p_flat = p_tables.reshape(5 * TP, D)

    deg_flat, g_rows = _sc_deg_time(sd_idx, tidx, p_flat, ones128)
    deg2 = deg_flat.reshape(NC, NPAD, D)

    h0 = pl.pallas_call(
        _tc_h0,
        out_shape=jax.ShapeDtypeStruct((NPAD, D), f32),
    )(x0_base, g_rows[:5 * N_INTER], deg2, W0)

    parts1 = _sc_propagate(h0, sd_idx).reshape(NC, NPAD, D)

    x0, rdegb = pl.pallas_call(
        _tc_x0_rdegb,
        out_shape=(jax.ShapeDtypeStruct((NPAD, D), f32),
                   jax.ShapeDtypeStruct((NPAD, D), f32)),
    )(x0_base, g_rows[:5 * N_INTER], deg2)

    x1, h1 = pl.pallas_call(
        _tc_x1_h1,
        grid=(8,),
        in_specs=[
            pl.BlockSpec((NC, _BLK, D), lambda i: (0, i, 0)),
            pl.BlockSpec((_BLK, D), lambda i: (i, 0)),
            pl.BlockSpec((1, D), lambda i: (0, 0)),
            pl.BlockSpec((D, D), lambda i: (0, 0)),
        ],
        out_specs=[
            pl.BlockSpec((_BLK, D), lambda i: (i, 0)),
            pl.BlockSpec((_BLK, D), lambda i: (i, 0)),
        ],
        out_shape=(jax.ShapeDtypeStruct((NPAD, D), f32),
                   jax.ShapeDtypeStruct((NPAD, D), f32)),
    )(parts1, rdegb, b0.reshape(1, D), W1)

    parts2 = _sc_propagate(h1, sd_idx).reshape(NC, NPAD, D)

    out = pl.pallas_call(
        _tc_final,
        grid=(10,),
        in_specs=[
            pl.BlockSpec((NC, 1000, D), lambda i: (0, i, 0)),
            pl.BlockSpec((1000, D), lambda i: (i, 0)),
            pl.BlockSpec((1, D), lambda i: (0, 0)),
            pl.BlockSpec((1000, D), lambda i: (i, 0)),
            pl.BlockSpec((1000, D), lambda i: (i, 0)),
        ],
        out_specs=pl.BlockSpec((1000, D), lambda i: (i, 0)),
        out_shape=jax.ShapeDtypeStruct((N_NODES, D), f32),
    )(parts2, rdegb, b1.reshape(1, D), x0, x1)
    return out


# speedup vs baseline: 1.0103x; 1.0103x over previous
import functools

import jax
import jax.numpy as jnp
from jax import lax
from jax.experimental import pallas as pl
from jax.experimental.pallas import tpu as pltpu
from jax.experimental.pallas import tpu_sc as plsc

N_USER, N_ITEM, N_ATT, N_INTER = 3000, 5000, 1000, 1000
N_NODES = N_USER + N_ITEM + N_ATT + N_INTER
E = 320000
D = 128
NPAD = 10112
NC, NS = 2, 16
NW = NC * NS
CH = 120
CW = 84
EPW = E // NW
ROWS_PT = NPAD // NS
TP = 96

_mesh = plsc.VectorSubcoreMesh(core_axis_name="c", subcore_axis_name="s")


def _zero_rows(zbuf, zrows, acc_sh, s):
    zero16 = jnp.zeros((16,), jnp.float32)

    def zbody(i, _):
        for g in range(D // 16):
            zbuf[i, pl.ds(g * 16, 16)] = zero16
        return 0
    lax.fori_loop(0, zrows, zbody, 0, unroll=False)
    full, rem = ROWS_PT // zrows, ROWS_PT % zrows
    for j in range(full):
        pltpu.sync_copy(zbuf.at[pl.ds(0, zrows)],
                        acc_sh.at[pl.ds(s * ROWS_PT + j * zrows, zrows)])
    if rem:
        pltpu.sync_copy(zbuf.at[pl.ds(0, rem)],
                        acc_sh.at[pl.ds(s * ROWS_PT + full * zrows, rem)])


@functools.partial(
    pl.kernel,
    out_type=(
        jax.ShapeDtypeStruct((NC * NPAD, D), jnp.float32),
        jax.ShapeDtypeStruct((NW * 160, D), jnp.float32),
    ),
    mesh=_mesh,
    scratch_types=(
        pltpu.VMEM((CW, 2, CH), jnp.int32),
        pltpu.VMEM((2, 80), jnp.int32),
        pltpu.VMEM((CH, D), jnp.float32),
        pltpu.VMEM((80, D), jnp.float32),
        pltpu.VMEM_SHARED((NPAD, D), jnp.float32),
        pltpu.SemaphoreType.DMA,
        pltpu.SemaphoreType.DMA,
    ),
)
def _sc_deg_time(sd_hbm, tidx_hbm, p_hbm, ones_hbm,
                 deg_out, g_out, idxv, tiv, onesv, gbuf, deg_sh,
                 scsem, sem):
    c = lax.axis_index("c")
    s = lax.axis_index("s")
    wid = s * NC + c

    pltpu.sync_copy(tidx_hbm.at[wid], tiv)
    pltpu.sync_copy(ones_hbm, onesv)
    _zero_rows(gbuf, 80, deg_sh, s)
    pltpu.sync_copy(sd_hbm.at[wid], idxv)
    plsc.subcore_barrier()

    def issue(k, _):
        pltpu.async_copy(onesv, deg_sh.at[idxv.at[k, 1]], scsem, add=True)
        return 0
    lax.fori_loop(0, CW, issue, 0, unroll=False)

    for j in range(2):
        pltpu.async_copy(p_hbm.at[tiv.at[j]], gbuf, sem).wait()
        pltpu.sync_copy(gbuf, g_out.at[pl.ds(wid * 160 + j * 80, 80)])

    def drain(k, _):
        pltpu.make_async_copy(onesv, deg_sh.at[idxv.at[k, 1]], scsem).wait()
        return 0
    lax.fori_loop(0, CW, drain, 0, unroll=False)

    plsc.subcore_barrier()
    pltpu.sync_copy(deg_sh.at[pl.ds(s * ROWS_PT, ROWS_PT)],
                    deg_out.at[pl.ds(c * NPAD + s * ROWS_PT, ROWS_PT)])


@functools.partial(
    pl.kernel,
    out_type=jax.ShapeDtypeStruct((NC * NPAD, D), jnp.float32),
    mesh=_mesh,
    scratch_types=(
        pltpu.VMEM((6, 2, CH), jnp.int32),
        pltpu.VMEM((3, CH, D), jnp.float32),
        pltpu.VMEM_SHARED((NPAD, D), jnp.float32),
        [pltpu.SemaphoreType.DMA] * 6,
        [pltpu.SemaphoreType.DMA] * 3,
        [pltpu.SemaphoreType.DMA] * 3,
    ),
)
def _sc_propagate(h_hbm, sd_hbm,
                  parts_out, idxv, rbuf, acc_sh, isems, gsems, scsems):
    c = lax.axis_index("c")
    s = lax.axis_index("s")
    wid = s * NC + c
    _zero_rows(rbuf.at[0], CH, acc_sh, s)
    for t in range(3):
        pltpu.async_copy(sd_hbm.at[wid, t], idxv.at[t], isems[t])
    plsc.subcore_barrier()

    def gather2(s6, b):
        pltpu.async_copy(h_hbm.at[idxv.at[s6, 0]], rbuf.at[b], gsems[b])

    def wait_gather2(s6, b):
        pltpu.make_async_copy(h_hbm.at[idxv.at[s6, 0]],
                              rbuf.at[b], gsems[b]).wait()

    def scatter2(s6, b):
        pltpu.async_copy(rbuf.at[b],
                         acc_sh.at[idxv.at[s6, 1]], scsems[b], add=True)

    def wait_scatter2(s6, b):
        pltpu.make_async_copy(rbuf.at[b],
                              acc_sh.at[idxv.at[s6, 1]], scsems[b]).wait()

    def sub_step(k, t, full):
        s6, b = t % 6, t % 3
        pltpu.make_async_copy(sd_hbm.at[wid, 0], idxv.at[s6],
                              isems[s6]).wait()
        if full or t >= 3:
            wait_scatter2(s6, b)
        gather2(s6, b)
        pltpu.async_copy(sd_hbm.at[wid, lax.rem(k + 3, CW)],
                         idxv.at[(s6 + 3) % 6], isems[(s6 + 3) % 6])
        if full or t >= 2:
            b2, s62 = (t + 1) % 3, (t + 4) % 6
            wait_gather2(s62, b2)
            scatter2(s62, b2)

    for t in range(6):
        sub_step(jnp.int32(t), t, full=False)

    def sextet(j, _):
        kb = 6 + 6 * j
        for t in range(6):
            sub_step(kb + t, t, full=True)
        return 0
    lax.fori_loop(0, (CW - 6) // 6, sextet, 0, unroll=False)

    for kv in (CW - 2, CW - 1):
        b2, s62 = kv % 3, kv % 6
        wait_gather2(s62, b2)
        scatter2(s62, b2)
    for b in range(3):
        wait_scatter2(b, b)
    for t in range(3):
        pltpu.make_async_copy(sd_hbm.at[wid, 0], idxv.at[t],
                              isems[t]).wait()

    plsc.subcore_barrier()
    pltpu.sync_copy(acc_sh.at[pl.ds(s * ROWS_PT, ROWS_PT)],
                    parts_out.at[pl.ds(c * NPAD + s * ROWS_PT, ROWS_PT)])


def _tc_ptables(te_ref, tth_ref, out_ref):
    for t in range(5):
        out_ref[t] = jnp.dot(te_ref[...], tth_ref[t],
                             preferred_element_type=jnp.float32,
                             precision=lax.Precision.HIGHEST)


def _rdeg(deg_ref):
    deg = deg_ref[0, :, 0:1] + deg_ref[1, :, 0:1]
    return lax.rsqrt(jnp.maximum(deg, 1.0))


def _assemble_x0(x0b_ref, g_ref):
    g = g_ref[...]
    x0t = (g[0:1000] + g[1000:2000] + g[2000:3000]
           + g[3000:4000] + g[4000:5000])
    x0 = x0b_ref[...]
    return jnp.concatenate(
        [x0[:N_USER + N_ITEM + N_ATT],
         x0[N_USER + N_ITEM + N_ATT:N_NODES] + x0t,
         x0[N_NODES:]], axis=0)


def _tc_x0_h0(x0b_ref, g_ref, deg_ref, w_ref, x0_ref, h0_ref, rdegb_ref):
    x0 = _assemble_x0(x0b_ref, g_ref)
    x0_ref[...] = x0
    rdeg = _rdeg(deg_ref)
    rdegb_ref[...] = jnp.broadcast_to(rdeg, (NPAD, D))
    h0_ref[...] = jnp.dot(x0, w_ref[...],
                          preferred_element_type=jnp.float32,
                          precision=lax.Precision.HIGHEST) * rdeg


def _leaky_next(parts, rdeg, b, row0, nrows):
    x = rdeg * (parts[0] + parts[1]) + b
    x = jnp.where(x >= 0, x, 0.01 * x)
    mask = (row0 + lax.broadcasted_iota(jnp.int32, (nrows, 1), 0)) < N_NODES
    return jnp.where(mask, x, 0.0)


_BLK = NPAD // 8


def _tc_x1_h1(parts_ref, rdegb_ref, b_ref, w_ref, x1_ref, h1_ref):
    i = pl.program_id(0)
    rdeg = rdegb_ref[...]
    x1 = _leaky_next(parts_ref[...], rdeg, b_ref[...], i * _BLK, _BLK)
    x1_ref[...] = x1
    h1_ref[...] = jnp.dot(x1, w_ref[...],
                          preferred_element_type=jnp.float32,
                          precision=lax.Precision.HIGHEST) * rdeg


def _tc_final(parts_ref, rdegb_ref, b_ref, x0_ref, x1_ref, out_ref):
    i = pl.program_id(0)
    x2 = _leaky_next(parts_ref[...], rdegb_ref[...], b_ref[...],
                     i * 1000, 1000)
    out_ref[...] = (x0_ref[...] + x1_ref[...] + x2) * (1.0 / 3.0)


def kernel(edge_index, user_embed, item_embed, att_embed, time_embed,
           time_node, time_to_hidden, W0, b0, W1, b1):
    f32, i32 = jnp.float32, jnp.int32

    src = edge_index[0].astype(i32).reshape(NW, EPW)
    dst = edge_index[1].astype(i32).reshape(NW, EPW)
    npad_e = CW * CH - EPW
    pad_idx = (N_NODES + (jnp.arange(npad_e, dtype=i32) % (NPAD - N_NODES)))
    pad_tile = jnp.tile(pad_idx[None], (NW, 1))
    src_idx = jnp.concatenate([src, pad_tile], axis=1).reshape(NW, CW, CH)
    dst_idx = jnp.concatenate([dst, pad_tile], axis=1).reshape(NW, CW, CH)
    sd_idx = jnp.stack([src_idx, dst_idx], axis=2)

    tn = time_node.astype(i32)
    tidx = (tn.T + TP * jnp.arange(5, dtype=i32)[:, None]).reshape(-1)
    pad_t = 94 + TP * (jnp.arange(NW * 160 - 5 * N_INTER, dtype=i32) % 5)
    tidx = jnp.concatenate([tidx, pad_t]).reshape(NW, 2, 80)

    te_pad = jnp.zeros((TP, D), f32).at[:94, :16].set(time_embed)
    tth_pad = jnp.zeros((5, D, D), f32).at[:, :16, :].set(
        time_to_hidden.reshape(5, 16, D))

    ones128 = jnp.ones((CH, D), f32)

    x0_base = jnp.concatenate(
        [user_embed, item_embed, att_embed,
         jnp.zeros((NPAD - N_USER - N_ITEM - N_ATT, D), f32)], axis=0)

    p_tables = pl.pallas_call(
        _tc_ptables,
        out_shape=jax.ShapeDtypeStruct((5, TP, D), f32),
    )(te_pad, tth_pad)
    p_flat = p_tables.reshape(5 * TP, D)

    deg_flat, g_rows = _sc_deg_time(sd_idx, tidx, p_flat, ones128)
    deg2 = deg_flat.reshape(NC, NPAD, D)

    x0, h0, rdegb = pl.pallas_call(
        _tc_x0_h0,
        out_shape=(jax.ShapeDtypeStruct((NPAD, D), f32),
                   jax.ShapeDtypeStruct((NPAD, D), f32),
                   jax.ShapeDtypeStruct((NPAD, D), f32)),
    )(x0_base, g_rows[:5 * N_INTER], deg2, W0)

    parts1 = _sc_propagate(h0, sd_idx).reshape(NC, NPAD, D)

    x1, h1 = pl.pallas_call(
        _tc_x1_h1,
        grid=(8,),
        in_specs=[
            pl.BlockSpec((NC, _BLK, D), lambda i: (0, i, 0)),
            pl.BlockSpec((_BLK, D), lambda i: (i, 0)),
            pl.BlockSpec((1, D), lambda i: (0, 0)),
            pl.BlockSpec((D, D), lambda i: (0, 0)),
        ],
        out_specs=[
            pl.BlockSpec((_BLK, D), lambda i: (i, 0)),
            pl.BlockSpec((_BLK, D), lambda i: (i, 0)),
        ],
        out_shape=(jax.ShapeDtypeStruct((NPAD, D), f32),
                   jax.ShapeDtypeStruct((NPAD, D), f32)),
    )(parts1, rdegb, b0.reshape(1, D), W1)

    parts2 = _sc_propagate(h1, sd_idx).reshape(NC, NPAD, D)

    out = pl.pallas_call(
        _tc_final,
        grid=(10,),
        in_specs=[
            pl.BlockSpec((NC, 1000, D), lambda i: (0, i, 0)),
            pl.BlockSpec((1000, D), lambda i: (i, 0)),
            pl.BlockSpec((1, D), lambda i: (0, 0)),
            pl.BlockSpec((1000, D), lambda i: (i, 0)),
            pl.BlockSpec((1000, D), lambda i: (i, 0)),
        ],
        out_specs=pl.BlockSpec((1000, D), lambda i: (i, 0)),
        out_shape=jax.ShapeDtypeStruct((N_NODES, D), f32),
    )(parts2, rdegb, b1.reshape(1, D), x0, x1)
    return out
